# Initial kernel scaffold; baseline (speedup 1.0000x reference)
import jax
import jax.numpy as jnp
from jax.experimental import pallas as pl


def kernel(x, financial_feat, write_idx, W_fin, b_fin, theta, W_enc, b_enc, W_proj, b_proj, bank_keys, bank_values):
    raise NotImplementedError("write your pallas kernel here")


# speedup vs baseline: 1.1431x; 1.1431x over previous
import functools
import math

import jax
import jax.numpy as jnp
from jax.experimental import pallas as pl
from jax.experimental.pallas import tpu as pltpu

B = 1024
DIM = 128
FIN = 4
MAX_DEPTH = 2

_INV_SQRT2 = 1.0 / math.sqrt(2.0)


def _gelu_exact(t):
    return 0.5 * t * (1.0 + jax.lax.erf(t * _INV_SQRT2))


def _core_kernel(x_ref, ff_ref, wfin_ref, bfin_ref, theta_ref, wenc_ref,
                 benc_ref, wproj_ref, bproj_ref, out_ref):
    x = x_ref[...]
    ff = ff_ref[...]
    b_fin = bfin_ref[...]
    b_enc = benc_ref[...]
    b_proj = bproj_ref[...]
    theta = theta_ref[...]
    w_enc = wenc_ref[...]
    w_proj = wproj_ref[...]

    fin = b_fin
    for c in range(FIN):
        fin = fin + ff[:, c:c + 1] * wfin_ref[c:c + 1, :]

    for _ in range(MAX_DEPTH):
        x = x + fin
        enc = jnp.maximum(
            jnp.dot(x, w_enc, preferred_element_type=jnp.float32) + b_enc, 0.0)
        x = _gelu_exact(jnp.dot(x, theta, preferred_element_type=jnp.float32))
        x = x + jnp.dot(enc, w_proj, preferred_element_type=jnp.float32) + b_proj

    out_ref[...] = x


@functools.partial(jax.jit, static_argnames=())
def kernel(x, financial_feat, write_idx, W_fin, b_fin, theta, W_enc, b_enc,
           W_proj, b_proj, bank_keys, bank_values):
    del write_idx, bank_keys, bank_values
    return pl.pallas_call(
        _core_kernel,
        out_shape=jax.ShapeDtypeStruct((B, DIM), jnp.float32),
        compiler_params=pltpu.CompilerParams(
            dimension_semantics=(),
        ),
    )(x, financial_feat, W_fin, b_fin.reshape(1, DIM), theta, W_enc,
      b_enc.reshape(1, DIM), W_proj, b_proj.reshape(1, DIM))
